# baseline (speedup 1.0000x reference)
import functools

import jax
import jax.numpy as jnp
from jax import lax
from jax.experimental import pallas as pl
from jax.experimental.pallas import tpu as pltpu
from jax.experimental.pallas import tpu_sc as plsc

NC = 2
NS = 16
NW = NC * NS
L = 16

BATCH = 16384
EMB = 64
ROWS_PER_W = BATCH // NW
CHUNK = 128
NCHUNK = ROWS_PER_W // CHUNK


def _sc_gather_partial():
    mesh = plsc.VectorSubcoreMesh(core_axis_name="c", subcore_axis_name="s")

    @functools.partial(
        pl.kernel,
        mesh=mesh,
        compiler_params=pltpu.CompilerParams(use_tc_tiling_on_sc=False),
        out_type=[
            jax.ShapeDtypeStruct((NW, L), jnp.float32),
            jax.ShapeDtypeStruct((BATCH // CHUNK, CHUNK), jnp.float32),
        ],
        scratch_types=[
            pltpu.VMEM((NCHUNK, CHUNK), jnp.int32),
            pltpu.VMEM((NCHUNK, CHUNK), jnp.int32),
            pltpu.VMEM((NCHUNK, CHUNK, EMB), jnp.float32),
            pltpu.VMEM((NCHUNK, CHUNK, EMB), jnp.float32),
            pltpu.VMEM((NCHUNK, CHUNK), jnp.float32),
            pltpu.VMEM((NCHUNK, CHUNK), jnp.float32),
            pltpu.VMEM((NCHUNK, CHUNK), jnp.float32),
            pltpu.VMEM((L,), jnp.float32),
            pltpu.SemaphoreType.DMA,
            pltpu.SemaphoreType.DMA,
            pltpu.SemaphoreType.DMA,
            pltpu.SemaphoreType.DMA,
        ],
    )
    def k(uidx_hbm, bidx_hbm, uemb_hbm, ubias_hbm, bemb_hbm, bbias_hbm,
          partials_hbm, bsum_hbm,
          uidx_v, bidx_v, urows_v, brows_v, ubv_v, bbv_v, bsum_v, acc_v,
          sem_eu, sem_eb, sem_bu, sem_bb):
        wid = lax.axis_index("s") * NC + lax.axis_index("c")
        row0 = wid * NCHUNK

        pltpu.sync_copy(uidx_hbm.at[pl.ds(row0, NCHUNK), :], uidx_v)
        pltpu.sync_copy(bidx_hbm.at[pl.ds(row0, NCHUNK), :], bidx_v)

        cps_u = []
        cps_b = []
        cps_bias = []
        for j in range(NCHUNK):
            cps_u.append(pltpu.async_copy(
                uemb_hbm.at[uidx_v.at[j]], urows_v.at[j], sem_eu))
            cps_b.append(pltpu.async_copy(
                bemb_hbm.at[bidx_v.at[j]], brows_v.at[j], sem_eb))
        for j in range(NCHUNK):
            cps_bias.append(pltpu.async_copy(
                ubias_hbm.at[uidx_v.at[j]], ubv_v.at[j], sem_bu))
            cps_bias.append(pltpu.async_copy(
                bbias_hbm.at[bidx_v.at[j]], bbv_v.at[j], sem_bb))

        for cp in cps_bias:
            cp.wait()
        for j in range(NCHUNK):
            for kk in range(CHUNK // L):
                s = pl.ds(kk * L, L)
                bsum_v[j, s] = ubv_v[j, s] + bbv_v[j, s]
        pltpu.sync_copy(bsum_v, bsum_hbm.at[pl.ds(row0, NCHUNK), :])

        zero = jnp.zeros((L,), jnp.float32)
        accs = (zero, zero, zero, zero)
        for j in range(NCHUNK):
            cps_u[j].wait()
            cps_b[j].wait()

            def body(rr, accs, j=j):
                a0, a1, a2, a3 = accs
                a0 = a0 + urows_v[j, rr, pl.ds(0, L)] * brows_v[j, rr, pl.ds(0, L)]
                a1 = a1 + urows_v[j, rr, pl.ds(L, L)] * brows_v[j, rr, pl.ds(L, L)]
                a2 = a2 + urows_v[j, rr, pl.ds(2 * L, L)] * brows_v[j, rr, pl.ds(2 * L, L)]
                a3 = a3 + urows_v[j, rr, pl.ds(3 * L, L)] * brows_v[j, rr, pl.ds(3 * L, L)]
                return a0, a1, a2, a3

            accs = lax.fori_loop(0, CHUNK, body, accs)

        acc_v[...] = (accs[0] + accs[1]) + (accs[2] + accs[3])
        pltpu.sync_copy(acc_v, partials_hbm.at[wid])

    return k


def _tc_finalize(partials, bias_sum):
    def body(p_ref, b_ref, o_ref):
        s = jnp.sum(p_ref[...])
        o_ref[...] = jax.nn.sigmoid(b_ref[...] + s)

    return pl.pallas_call(
        body,
        out_shape=jax.ShapeDtypeStruct(bias_sum.shape, jnp.float32),
    )(partials, bias_sum)


def kernel(inputs, user_embedding, user_bias, book_embedding, book_bias):
    uidx = inputs[:, 0].reshape(BATCH // CHUNK, CHUNK)
    bidx = inputs[:, 1].reshape(BATCH // CHUNK, CHUNK)
    ub = user_bias.reshape(-1)
    bb = book_bias.reshape(-1)

    partials, bsum = _sc_gather_partial()(
        uidx, bidx, user_embedding, ub, book_embedding, bb)
    out = _tc_finalize(partials, bsum)
    return out.reshape(BATCH, 1)


# speedup vs baseline: 1.3947x; 1.3947x over previous
import functools

import jax
import jax.numpy as jnp
from jax import lax
from jax.experimental import pallas as pl
from jax.experimental.pallas import tpu as pltpu
from jax.experimental.pallas import tpu_sc as plsc

NC = 2
NS = 16
NW = NC * NS
L = 16

BATCH = 16384
EMB = 64
ROWS_PER_W = BATCH // NW
PASS_ROWS = 128
NPASS = ROWS_PER_W // PASS_ROWS
PASS_GROUPS = PASS_ROWS // L


def _sc_gather_partial():
    mesh = plsc.VectorSubcoreMesh(core_axis_name="c", subcore_axis_name="s")

    @functools.partial(
        pl.kernel,
        mesh=mesh,
        out_type=[
            jax.ShapeDtypeStruct((NW * L,), jnp.float32),
            jax.ShapeDtypeStruct((BATCH,), jnp.float32),
        ],
        scratch_types=[
            pltpu.VMEM((ROWS_PER_W,), jnp.int32),
            pltpu.VMEM((ROWS_PER_W,), jnp.int32),
            pltpu.VMEM((PASS_ROWS, EMB), jnp.float32),
            pltpu.VMEM((PASS_ROWS, EMB), jnp.float32),
            pltpu.VMEM((PASS_ROWS, EMB), jnp.float32),
            pltpu.VMEM((PASS_ROWS, EMB), jnp.float32),
            pltpu.VMEM((ROWS_PER_W,), jnp.float32),
            pltpu.VMEM((ROWS_PER_W,), jnp.float32),
            pltpu.VMEM((ROWS_PER_W,), jnp.float32),
            pltpu.VMEM((L,), jnp.float32),
            pltpu.SemaphoreType.DMA,
            pltpu.SemaphoreType.DMA,
            pltpu.SemaphoreType.DMA,
            pltpu.SemaphoreType.DMA,
        ],
    )
    def k(uidx_hbm, bidx_hbm, uemb_hbm, ubias_hbm, bemb_hbm, bbias_hbm,
          partials_hbm, bsum_hbm,
          uidx_v, bidx_v, ur0, ur1, br0, br1, ubv_v, bbv_v, bsum_v, acc_v,
          sem_r0, sem_r1, sem_bu, sem_bb):
        wid = lax.axis_index("s") * NC + lax.axis_index("c")
        base = wid * ROWS_PER_W

        ubufs = (ur0, ur1)
        bbufs = (br0, br1)
        sems = (sem_r0, sem_r1)

        pltpu.sync_copy(uidx_hbm.at[pl.ds(base, ROWS_PER_W)], uidx_v)
        pltpu.sync_copy(bidx_hbm.at[pl.ds(base, ROWS_PER_W)], bidx_v)

        cp_bu = pltpu.async_copy(ubias_hbm.at[uidx_v], ubv_v, sem_bu)
        cp_bb = pltpu.async_copy(bbias_hbm.at[bidx_v], bbv_v, sem_bb)

        def enqueue_pass(p, ubuf, bbuf, sem):
            def enq(g, _):
                uvec = uidx_v[pl.ds(p * PASS_ROWS + g * L, L)]
                bvec = bidx_v[pl.ds(p * PASS_ROWS + g * L, L)]
                for j in range(L):
                    iu = uvec[j]
                    pltpu.async_copy(
                        uemb_hbm.at[pl.ds(iu, 1), :],
                        ubuf.at[pl.ds(g * L + j, 1), :], sem)
                    ib = bvec[j]
                    pltpu.async_copy(
                        bemb_hbm.at[pl.ds(ib, 1), :],
                        bbuf.at[pl.ds(g * L + j, 1), :], sem)
                return 0

            lax.fori_loop(0, PASS_GROUPS, enq, 0)

        def drain_pass(ubuf, sem):
            def dr(g, _):
                for _j in range(2 * L):
                    pltpu.make_async_copy(
                        uemb_hbm.at[pl.ds(0, 1), :],
                        ubuf.at[pl.ds(0, 1), :], sem).wait()
                return 0

            lax.fori_loop(0, PASS_GROUPS, dr, 0)

        def compute_pass(ubuf, bbuf, accs):
            def body(rr, accs):
                a0, a1, a2, a3 = accs
                a0 = a0 + ubuf[rr, pl.ds(0, L)] * bbuf[rr, pl.ds(0, L)]
                a1 = a1 + ubuf[rr, pl.ds(L, L)] * bbuf[rr, pl.ds(L, L)]
                a2 = a2 + ubuf[rr, pl.ds(2 * L, L)] * bbuf[rr, pl.ds(2 * L, L)]
                a3 = a3 + ubuf[rr, pl.ds(3 * L, L)] * bbuf[rr, pl.ds(3 * L, L)]
                return a0, a1, a2, a3

            return lax.fori_loop(0, PASS_ROWS, body, accs)

        enqueue_pass(0, ubufs[0], bbufs[0], sems[0])

        zero = jnp.zeros((L,), jnp.float32)
        accs = (zero, zero, zero, zero)
        for p in range(NPASS):
            if p + 1 < NPASS:
                enqueue_pass(p + 1, ubufs[(p + 1) % 2], bbufs[(p + 1) % 2],
                             sems[(p + 1) % 2])
            drain_pass(ubufs[p % 2], sems[p % 2])
            accs = compute_pass(ubufs[p % 2], bbufs[p % 2], accs)

        cp_bu.wait()
        cp_bb.wait()
        for kk in range(ROWS_PER_W // L):
            s = pl.ds(kk * L, L)
            bsum_v[s] = ubv_v[s] + bbv_v[s]
        pltpu.sync_copy(bsum_v, bsum_hbm.at[pl.ds(base, ROWS_PER_W)])

        acc_v[...] = (accs[0] + accs[1]) + (accs[2] + accs[3])
        pltpu.sync_copy(acc_v, partials_hbm.at[pl.ds(wid * L, L)])

    return k


def _tc_finalize(partials, bias_sum):
    def body(p_ref, b_ref, o_ref):
        s = jnp.sum(p_ref[...])
        o_ref[...] = jax.nn.sigmoid(b_ref[...] + s)

    return pl.pallas_call(
        body,
        out_shape=jax.ShapeDtypeStruct(bias_sum.shape, jnp.float32),
    )(partials, bias_sum)


def kernel(inputs, user_embedding, user_bias, book_embedding, book_bias):
    uidx = inputs[:, 0]
    bidx = inputs[:, 1]
    ub = user_bias.reshape(-1)
    bb = book_bias.reshape(-1)

    partials, bsum = _sc_gather_partial()(
        uidx, bidx, user_embedding, ub, book_embedding, bb)
    out = _tc_finalize(partials.reshape(NW, L),
                       bsum.reshape(BATCH // 128, 128))
    return out.reshape(BATCH, 1)
